# baseline (speedup 1.0000x reference)
import jax
import jax.numpy as jnp
from jax.experimental import pallas as pl
from jax.experimental.pallas import tpu as pltpu


def _mse_kernel(p_ref, t_ref, m_ref, out_ref, acc_ref, cnt_ref):
    i = pl.program_id(0)

    @pl.when(i == 0)
    def _init():
        acc_ref[...] = jnp.zeros_like(acc_ref)
        cnt_ref[...] = jnp.zeros_like(cnt_ref)

    m = m_ref[...]
    d = jnp.where(m, 0.0, p_ref[...] - t_ref[...])
    acc_ref[...] += d * d
    cnt_ref[...] += jnp.where(m, 0, 1)

    @pl.when(i == pl.num_programs(0) - 1)
    def _fin():
        loss = jnp.sum(acc_ref[...])
        count = jnp.sum(cnt_ref[...].astype(jnp.float32))
        out_ref[...] = jnp.full((1, 1), loss / count, jnp.float32)


def kernel(preds, target, mask):
    B, T = preds.shape
    BR = 256
    out = pl.pallas_call(
        _mse_kernel,
        grid=(B // BR,),
        in_specs=[
            pl.BlockSpec((BR, T), lambda i: (i, 0)),
            pl.BlockSpec((BR, T), lambda i: (i, 0)),
            pl.BlockSpec((BR, T), lambda i: (i, 0)),
        ],
        out_specs=pl.BlockSpec((1, 1), lambda i: (0, 0)),
        out_shape=jax.ShapeDtypeStruct((1, 1), jnp.float32),
        scratch_shapes=[
            pltpu.VMEM((BR, T), jnp.float32),
            pltpu.VMEM((BR, T), jnp.int32),
        ],
    )(preds, target, mask)
    return out[0, 0]


# speedup vs baseline: 1.0270x; 1.0270x over previous
import jax
import jax.numpy as jnp
from jax.experimental import pallas as pl
from jax.experimental.pallas import tpu as pltpu


def _mse_kernel(p_ref, t_ref, m_ref, out_ref, acc_ref, cnt_ref):
    i = pl.program_id(0)

    @pl.when(i == 0)
    def _init():
        acc_ref[...] = jnp.zeros_like(acc_ref)
        cnt_ref[...] = jnp.zeros_like(cnt_ref)

    m = m_ref[...]
    d = jnp.where(m, 0.0, p_ref[...] - t_ref[...])
    acc_ref[...] += d * d
    cnt_ref[...] += jnp.where(m, 0, 1)

    @pl.when(i == pl.num_programs(0) - 1)
    def _fin():
        loss = jnp.sum(acc_ref[...])
        count = jnp.sum(cnt_ref[...].astype(jnp.float32))
        out_ref[...] = jnp.full((1, 1), loss / count, jnp.float32)


def kernel(preds, target, mask):
    B, T = preds.shape
    BR = 512
    out = pl.pallas_call(
        _mse_kernel,
        grid=(B // BR,),
        in_specs=[
            pl.BlockSpec((BR, T), lambda i: (i, 0)),
            pl.BlockSpec((BR, T), lambda i: (i, 0)),
            pl.BlockSpec((BR, T), lambda i: (i, 0)),
        ],
        out_specs=pl.BlockSpec((1, 1), lambda i: (0, 0)),
        out_shape=jax.ShapeDtypeStruct((1, 1), jnp.float32),
        scratch_shapes=[
            pltpu.VMEM((BR, T), jnp.float32),
            pltpu.VMEM((BR, T), jnp.int32),
        ],
    )(preds, target, mask)
    return out[0, 0]


# speedup vs baseline: 1.0288x; 1.0018x over previous
import jax
import jax.numpy as jnp
from jax.experimental import pallas as pl
from jax.experimental.pallas import tpu as pltpu


def _mse_kernel(p_ref, t_ref, m_ref, out_ref, acc_ref, cnt_ref):
    i = pl.program_id(0)
    BR, T = p_ref.shape

    def body(r, carry):
        acc, cnt = carry
        s = r * 8
        m = m_ref[pl.ds(s, 8), :]
        d = jnp.where(m, 0.0, p_ref[pl.ds(s, 8), :] - t_ref[pl.ds(s, 8), :])
        acc = acc + d * d
        cnt = cnt + jnp.where(m, 0.0, 1.0)
        return acc, cnt

    z = jnp.zeros((8, T), jnp.float32)
    acc, cnt = jax.lax.fori_loop(0, BR // 8, body, (z, z))

    @pl.when(i == 0)
    def _init():
        acc_ref[...] = jnp.zeros_like(acc_ref)
        cnt_ref[...] = jnp.zeros_like(cnt_ref)

    acc_ref[...] += acc
    cnt_ref[...] += cnt

    @pl.when(i == pl.num_programs(0) - 1)
    def _fin():
        loss = jnp.sum(acc_ref[...])
        count = jnp.sum(cnt_ref[...])
        out_ref[...] = jnp.full((1, 1), loss / count, jnp.float32)


def kernel(preds, target, mask):
    B, T = preds.shape
    BR = 512
    out = pl.pallas_call(
        _mse_kernel,
        grid=(B // BR,),
        in_specs=[
            pl.BlockSpec((BR, T), lambda i: (i, 0)),
            pl.BlockSpec((BR, T), lambda i: (i, 0)),
            pl.BlockSpec((BR, T), lambda i: (i, 0)),
        ],
        out_specs=pl.BlockSpec((1, 1), lambda i: (0, 0)),
        out_shape=jax.ShapeDtypeStruct((1, 1), jnp.float32),
        scratch_shapes=[
            pltpu.VMEM((8, T), jnp.float32),
            pltpu.VMEM((8, T), jnp.float32),
        ],
    )(preds, target, mask)
    return out[0, 0]
